# baseline (speedup 1.0000x reference)
import jax
import jax.numpy as jnp
from jax import lax
from jax.experimental import pallas as pl
from jax.experimental.pallas import tpu as pltpu
from jax.experimental.pallas import tpu_sc as plsc

N = 10000
E = 320000
H = 128

NC = 2
NS = 16
NW = NC * NS
C = 128
K = 3
SS = K * C
T = 27
EP = NW * T * SS
NCH = EP // SS
NBO = N // 400
ZB = N // SS


def _sc_agg_kernel(x_hbm, idx_hbm, out0, out1, iall, rows_a, acc_sh,
                   sem_g, sem_s):
    c = lax.axis_index("c")
    s = lax.axis_index("s")
    wid = c * NS + s
    HS = SS

    def rbuf(k):
        return rows_a.at[pl.ds(k * C, C)]

    def zbody(r, _):
        def zcol(j, _):
            rows_a[r, pl.ds(j * 16, 16)] = jnp.zeros((16,), jnp.float32)
            return 0

        lax.fori_loop(0, H // 16, zcol, 0)
        return 0

    lax.fori_loop(0, HS, zbody, 0)
    NZB = N // HS
    for j in range(2):
        blk = s + j * NS

        @pl.when(blk < NZB)
        def _():
            off = pl.multiple_of(blk * HS, 8)
            pltpu.sync_copy(rows_a, acc_sh.at[pl.ds(off, HS)])

    @pl.when(s == NS - 1)
    def _():
        rem = N - NZB * HS
        pltpu.sync_copy(rows_a.at[pl.ds(0, rem)],
                        acc_sh.at[pl.ds(NZB * HS, rem)])

    plsc.subcore_barrier()

    cb = wid * T

    def body(t, _):
        pltpu.sync_copy(idx_hbm.at[cb + t], iall)
        for k in range(K):
            pltpu.async_copy(x_hbm.at[iall.at[k]], rbuf(k), sem_g)
        for k in range(K):
            pltpu.make_async_copy(x_hbm.at[iall.at[k]], rbuf(k), sem_g).wait()
        for k in range(K):
            pltpu.async_copy(rbuf(k), acc_sh.at[iall.at[K + k]],
                             sem_s, add=True)
        for k in range(K):
            pltpu.make_async_copy(rbuf(k), acc_sh.at[iall.at[K + k]],
                                  sem_s).wait()
        return 0

    lax.fori_loop(0, T, body, 0)
    plsc.subcore_barrier()

    for j in range(2):
        blk = s + j * NS

        @pl.when(blk < NBO)
        def _():
            off = pl.multiple_of(blk * 400, 8)
            sl = pl.ds(off, 400)

            @pl.when(c == 0)
            def _():
                pltpu.sync_copy(acc_sh.at[sl], out0.at[sl])

            @pl.when(c == 1)
            def _():
                pltpu.sync_copy(acc_sh.at[sl], out1.at[sl])


def _sc_agg(x, idx3):
    mesh = plsc.VectorSubcoreMesh(core_axis_name="c", subcore_axis_name="s",
                                  num_cores=NC, num_subcores=NS)
    f = pl.kernel(
        _sc_agg_kernel,
        out_type=(jax.ShapeDtypeStruct((N, H), jnp.float32),
                  jax.ShapeDtypeStruct((N, H), jnp.float32)),
        mesh=mesh,
        scratch_types=[
            pltpu.VMEM((2 * K, C), jnp.int32),
            pltpu.VMEM((SS, H), jnp.float32),
            pltpu.VMEM_SHARED((N + 8, H), jnp.float32),
            pltpu.SemaphoreType.DMA,
            pltpu.SemaphoreType.DMA,
        ],
    )
    return f(x, idx3)


def _pack_idx(gather_idx, scatter_idx):
    pad = EP - E
    g = jnp.concatenate([gather_idx, jnp.zeros((pad,), jnp.int32)])
    sc = jnp.concatenate([scatter_idx, jnp.full((pad,), N, jnp.int32)])
    g = g.reshape(NCH, K, C)
    sc = sc.reshape(NCH, K, C)
    return jnp.concatenate([g, sc], axis=1)


def _mlp_body(eps_ref, x_ref, a0_ref, a1_ref, W1_ref, g_ref, b_ref, W2_ref,
              lng_ref, lnb_ref, dir_ref, o_ref):
    x = x_ref[...]
    h = x * (1.0 + eps_ref[0]) + a0_ref[...] + a1_ref[...]
    h = jnp.dot(h, W1_ref[...], preferred_element_type=jnp.float32)
    m = jnp.mean(h, axis=-1, keepdims=True)
    v = jnp.mean((h - m) * (h - m), axis=-1, keepdims=True)
    h = (h - m) * lax.rsqrt(v + 1e-5) * g_ref[...] + b_ref[...]
    h = jnp.maximum(h, 0.0)
    h = jnp.dot(h, W2_ref[...], preferred_element_type=jnp.float32)
    y = jnp.maximum(h + x + dir_ref[...], 0.0)
    m2 = jnp.mean(y, axis=-1, keepdims=True)
    v2 = jnp.mean((y - m2) * (y - m2), axis=-1, keepdims=True)
    o_ref[...] = (y - m2) * lax.rsqrt(v2 + 1e-5) * lng_ref[...] + lnb_ref[...]


BN = 1000


def _mlp(x, a0, a1, eps, W1, g, b, W2, lng, lnb, dir_row):
    grid = (N // BN,)
    row_spec = pl.BlockSpec((BN, H), lambda i: (i, 0))
    full = lambda a: pl.BlockSpec(a.shape, lambda i: (0,) * a.ndim)
    g_, b_ = g.reshape(1, -1), b.reshape(1, -1)
    lng_, lnb_ = lng.reshape(1, -1), lnb.reshape(1, -1)
    dir_ = dir_row.reshape(1, -1)
    return pl.pallas_call(
        _mlp_body,
        grid=grid,
        in_specs=[
            pl.BlockSpec(memory_space=pltpu.SMEM),
            row_spec, row_spec, row_spec,
            full(W1), full(g_), full(b_), full(W2),
            full(lng_), full(lnb_), full(dir_),
        ],
        out_specs=row_spec,
        out_shape=jax.ShapeDtypeStruct((N, H), jnp.float32),
    )(eps.reshape(1), x, a0, a1, W1, g_, b_, W2, lng_, lnb_, dir_)


def kernel(x, edge_index, eps_d, W1_d, g_d, b_d, W2_d, eps_u, W1_u, g_u,
           b_u, W2_u, ln1_g, ln1_b, ln2_g, ln2_b, dir_emb):
    src = edge_index[0].astype(jnp.int32)
    dst = edge_index[1].astype(jnp.int32)
    idx_d = _pack_idx(src, dst)
    idx_u = _pack_idx(dst, src)
    a0, a1 = _sc_agg(x, idx_d)
    x1 = _mlp(x, a0, a1, eps_d, W1_d, g_d, b_d, W2_d, ln1_g, ln1_b, dir_emb[0])
    b0, b1 = _sc_agg(x1, idx_u)
    x2 = _mlp(x1, b0, b1, eps_u, W1_u, g_u, b_u, W2_u, ln2_g, ln2_b, dir_emb[1])
    return x2


# speedup vs baseline: 1.3818x; 1.3818x over previous
import jax
import jax.numpy as jnp
from jax import lax
from jax.experimental import pallas as pl
from jax.experimental.pallas import tpu as pltpu
from jax.experimental.pallas import tpu_sc as plsc

N = 10000
E = 320000
H = 128

NC = 2
NS = 16
NW = NC * NS
C = 80
K = 2
SS = K * C
T = 64
EP = NW * T * SS
NCH = EP // SS
NBO = N // 400
ZB = N // SS


def _sc_agg_kernel(x_hbm, idx_hbm, out0, out1, iall0, iall1, iall2,
                   rows0, rows1, acc_sh, sem_g, sem_s0, sem_s1,
                   sem_i0, sem_i1, sem_i2):
    c = lax.axis_index("c")
    s = lax.axis_index("s")
    wid = c * NS + s
    iall = (iall0, iall1, iall2)
    rows = (rows0, rows1)
    sem_s = (sem_s0, sem_s1)
    sem_i = (sem_i0, sem_i1, sem_i2)

    def zbody(r, _):
        def zcol(j, _):
            rows0[r, pl.ds(j * 16, 16)] = jnp.zeros((16,), jnp.float32)
            return 0

        lax.fori_loop(0, H // 16, zcol, 0)
        return 0

    lax.fori_loop(0, SS, zbody, 0)
    NZB = N // SS
    for j in range((NZB + NS - 1) // NS):
        blk = s + j * NS

        @pl.when(blk < NZB)
        def _():
            off = pl.multiple_of(blk * SS, 8)
            pltpu.sync_copy(rows0, acc_sh.at[pl.ds(off, SS)])

    @pl.when(s == NS - 1)
    def _():
        rem = N - NZB * SS
        pltpu.sync_copy(rows0.at[pl.ds(0, rem)],
                        acc_sh.at[pl.ds(NZB * SS, rem)])

    plsc.subcore_barrier()

    cb = wid * T

    def issue_idx(t, m):
        seg = pl.multiple_of((cb + t) * 2 * SS, 8)
        pltpu.async_copy(idx_hbm.at[pl.ds(seg, 2 * SS)], iall[m], sem_i[m])

    def drain_idx(m):
        seg0 = pl.multiple_of(cb * 2 * SS, 8)
        pltpu.make_async_copy(idx_hbm.at[pl.ds(seg0, 2 * SS)], iall[m],
                              sem_i[m]).wait()

    def issue_gather(S, m):
        for k in range(K):
            pltpu.async_copy(x_hbm.at[iall[m].at[pl.ds(k * C, C)]],
                             rows[S].at[pl.ds(k * C, C)], sem_g)

    def drain_gather(S, m):
        for k in range(K):
            pltpu.make_async_copy(x_hbm.at[iall[m].at[pl.ds(k * C, C)]],
                                  rows[S].at[pl.ds(k * C, C)], sem_g).wait()

    def issue_scatter(S, m):
        for k in range(K):
            pltpu.async_copy(rows[S].at[pl.ds(k * C, C)],
                             acc_sh.at[iall[m].at[pl.ds(SS + k * C, C)]],
                             sem_s[S], add=True)

    def drain_scatter(S, m):
        for k in range(K):
            pltpu.make_async_copy(rows[S].at[pl.ds(k * C, C)],
                                  acc_sh.at[iall[m].at[pl.ds(SS + k * C, C)]],
                                  sem_s[S]).wait()

    issue_idx(0, 0)
    issue_idx(1, 1)
    drain_idx(0)
    issue_gather(0, 0)

    def step(t, S, m):
        So, m1, m2 = 1 - S, (m + 1) % 3, (m + 2) % 3
        drain_gather(S, m)
        issue_scatter(S, m)

        @pl.when(t >= 1)
        def _():
            drain_scatter(So, m2)

        @pl.when(t + 2 < T)
        def _():
            issue_idx(t + 2, m2)

        @pl.when(t + 1 < T)
        def _():
            drain_idx(m1)
            issue_gather(So, m1)

    def body(t, _):
        for r in range(6):
            @pl.when(t % 6 == r)
            def _(r=r):
                step(t, r % 2, r % 3)

        return 0

    lax.fori_loop(0, T, body, 0)
    drain_scatter((T - 1) % 2, (T - 1) % 3)
    plsc.subcore_barrier()

    for j in range(2):
        blk = s + j * NS

        @pl.when(blk < NBO)
        def _():
            off = pl.multiple_of(blk * 400, 8)
            sl = pl.ds(off, 400)

            @pl.when(c == 0)
            def _():
                pltpu.sync_copy(acc_sh.at[sl], out0.at[sl])

            @pl.when(c == 1)
            def _():
                pltpu.sync_copy(acc_sh.at[sl], out1.at[sl])


def _sc_agg(x, idx3):
    mesh = plsc.VectorSubcoreMesh(core_axis_name="c", subcore_axis_name="s",
                                  num_cores=NC, num_subcores=NS)
    f = pl.kernel(
        _sc_agg_kernel,
        out_type=(jax.ShapeDtypeStruct((N, H), jnp.float32),
                  jax.ShapeDtypeStruct((N, H), jnp.float32)),
        mesh=mesh,
        scratch_types=[
            pltpu.VMEM((2 * SS,), jnp.int32),
            pltpu.VMEM((2 * SS,), jnp.int32),
            pltpu.VMEM((2 * SS,), jnp.int32),
            pltpu.VMEM((SS, H), jnp.float32),
            pltpu.VMEM((SS, H), jnp.float32),
            pltpu.VMEM_SHARED((N + 8, H), jnp.float32),
            pltpu.SemaphoreType.DMA,
            pltpu.SemaphoreType.DMA,
            pltpu.SemaphoreType.DMA,
            pltpu.SemaphoreType.DMA,
            pltpu.SemaphoreType.DMA,
            pltpu.SemaphoreType.DMA,
        ],
    )
    return f(x, idx3)


def _pack_idx(gather_idx, scatter_idx):
    pad = EP - E
    g = jnp.concatenate([gather_idx, jnp.zeros((pad,), jnp.int32)])
    sc = jnp.concatenate([scatter_idx, jnp.full((pad,), N, jnp.int32)])
    g = g.reshape(NCH, SS)
    sc = sc.reshape(NCH, SS)
    return jnp.concatenate([g, sc], axis=1).reshape(-1)


def _mlp_body(eps_ref, x_ref, a0_ref, a1_ref, W1_ref, g_ref, b_ref, W2_ref,
              lng_ref, lnb_ref, dir_ref, o_ref):
    x = x_ref[...]
    h = x * (1.0 + eps_ref[0]) + a0_ref[...] + a1_ref[...]
    h = jnp.dot(h, W1_ref[...], preferred_element_type=jnp.float32)
    m = jnp.mean(h, axis=-1, keepdims=True)
    v = jnp.mean((h - m) * (h - m), axis=-1, keepdims=True)
    h = (h - m) * lax.rsqrt(v + 1e-5) * g_ref[...] + b_ref[...]
    h = jnp.maximum(h, 0.0)
    h = jnp.dot(h, W2_ref[...], preferred_element_type=jnp.float32)
    y = jnp.maximum(h + x + dir_ref[...], 0.0)
    m2 = jnp.mean(y, axis=-1, keepdims=True)
    v2 = jnp.mean((y - m2) * (y - m2), axis=-1, keepdims=True)
    o_ref[...] = (y - m2) * lax.rsqrt(v2 + 1e-5) * lng_ref[...] + lnb_ref[...]


BN = 1000


def _mlp(x, a0, a1, eps, W1, g, b, W2, lng, lnb, dir_row):
    grid = (N // BN,)
    row_spec = pl.BlockSpec((BN, H), lambda i: (i, 0))
    full = lambda a: pl.BlockSpec(a.shape, lambda i: (0,) * a.ndim)
    g_, b_ = g.reshape(1, -1), b.reshape(1, -1)
    lng_, lnb_ = lng.reshape(1, -1), lnb.reshape(1, -1)
    dir_ = dir_row.reshape(1, -1)
    return pl.pallas_call(
        _mlp_body,
        grid=grid,
        in_specs=[
            pl.BlockSpec(memory_space=pltpu.SMEM),
            row_spec, row_spec, row_spec,
            full(W1), full(g_), full(b_), full(W2),
            full(lng_), full(lnb_), full(dir_),
        ],
        out_specs=row_spec,
        out_shape=jax.ShapeDtypeStruct((N, H), jnp.float32),
    )(eps.reshape(1), x, a0, a1, W1, g_, b_, W2, lng_, lnb_, dir_)


def kernel(x, edge_index, eps_d, W1_d, g_d, b_d, W2_d, eps_u, W1_u, g_u,
           b_u, W2_u, ln1_g, ln1_b, ln2_g, ln2_b, dir_emb):
    src = edge_index[0].astype(jnp.int32)
    dst = edge_index[1].astype(jnp.int32)
    idx_d = _pack_idx(src, dst)
    idx_u = _pack_idx(dst, src)
    a0, a1 = _sc_agg(x, idx_d)
    x1 = _mlp(x, a0, a1, eps_d, W1_d, g_d, b_d, W2_d, ln1_g, ln1_b, dir_emb[0])
    b0, b1 = _sc_agg(x1, idx_u)
    x2 = _mlp(x1, b0, b1, eps_u, W1_u, g_u, b_u, W2_u, ln2_g, ln2_b, dir_emb[1])
    return x2
